# baseline (speedup 1.0000x reference)
import functools

import jax
import jax.numpy as jnp
from jax import lax
from jax.experimental import pallas as pl
from jax.experimental.pallas import tpu as pltpu
from jax.experimental.pallas import tpu_sc as plsc

D_MODEL = 1024
_NUM_CORES = 2
_NUM_SUBCORES = 16
_NUM_WORKERS = _NUM_CORES * _NUM_SUBCORES


@functools.lru_cache(maxsize=None)
def _build_embedding_kernel(B: int, chunk: int, nbuf: int, lag: int = 2):
    rows_per_worker = B // _NUM_WORKERS
    n_chunks = rows_per_worker // chunk
    mesh = plsc.VectorSubcoreMesh(core_axis_name="c", subcore_axis_name="s")

    @functools.partial(
        pl.kernel,
        mesh=mesh,
        out_type=jax.ShapeDtypeStruct((B, D_MODEL), jnp.float32),
        scratch_types=[
            pltpu.VMEM((rows_per_worker,), jnp.int32),
            pltpu.VMEM((nbuf, chunk, D_MODEL), jnp.float32),
        ]
        + [pltpu.SemaphoreType.DMA] * (2 * nbuf),
    )
    def emb(toks_hbm, table_hbm, out_hbm, idx_v, buf_v, *sems):
        gsems = sems[:nbuf]
        osems = sems[nbuf:]
        wid = lax.axis_index("s") * _NUM_CORES + lax.axis_index("c")
        base = wid * rows_per_worker
        pltpu.sync_copy(toks_hbm.at[pl.ds(base, rows_per_worker)], idx_v)

        def gather_copy(ci, slot):
            return pltpu.make_async_copy(
                table_hbm.at[idx_v.at[pl.ds(ci * chunk, chunk)]],
                buf_v.at[slot],
                gsems[slot],
            )

        def out_copy(ci, slot):
            return pltpu.make_async_copy(
                buf_v.at[slot],
                out_hbm.at[pl.ds(base + ci * chunk, chunk)],
                osems[slot],
            )

        for i in range(n_chunks + lag):
            if i < n_chunks:
                s = i % nbuf
                if i >= nbuf:
                    out_copy(i - nbuf, s).wait()
                gather_copy(i, s).start()
            j = i - lag
            if j >= 0:
                gather_copy(j, j % nbuf).wait()
                out_copy(j, j % nbuf).start()
        for j in range(max(0, n_chunks - nbuf), n_chunks):
            out_copy(j, j % nbuf).wait()

    return emb


@functools.lru_cache(maxsize=None)
def _build_scs_gather(B: int, chunk: int):
    rows_per_core = B // _NUM_CORES
    n_chunks = rows_per_core // chunk
    mesh = plsc.ScalarSubcoreMesh(axis_name="c", num_cores=_NUM_CORES)

    @functools.partial(
        pl.kernel,
        mesh=mesh,
        out_type=jax.ShapeDtypeStruct((B, D_MODEL), jnp.float32),
        scratch_types=[
            pltpu.SMEM((chunk,), jnp.int32),
            pltpu.VMEM_SHARED((chunk, D_MODEL), jnp.float32),
            pltpu.SemaphoreType.DMA,
            pltpu.SemaphoreType.DMA,
        ],
    )
    def scs(toks_hbm, table_hbm, out_hbm, idx_s, buf_s, gsem, osem):
        cid = lax.axis_index("c")
        base = cid * rows_per_core

        def chunk_body(ci):
            start = base + ci * chunk
            pltpu.sync_copy(toks_hbm.at[pl.ds(start, chunk)], idx_s)

            def row_body(r):
                idx = idx_s[r]
                pltpu.make_async_copy(
                    table_hbm.at[pl.ds(idx, 1)],
                    buf_s.at[pl.ds(r, 1)],
                    gsem,
                ).start()

            pl.loop(0, chunk)(row_body)
            pltpu.make_async_copy(
                table_hbm.at[pl.ds(0, chunk)], buf_s, gsem
            ).wait()
            pltpu.sync_copy(buf_s, out_hbm.at[pl.ds(start, chunk)])

        pl.loop(0, n_chunks)(chunk_body)

    return scs


def kernel(toks, W_E):
    n_batch, seq = toks.shape
    B = n_batch * seq
    flat = toks.reshape(B).astype(jnp.int32)
    out = _build_scs_gather(B, 512)(flat, W_E)
    return out.reshape(n_batch, seq, D_MODEL)


# speedup vs baseline: 2.0588x; 2.0588x over previous
import functools

import jax
import jax.numpy as jnp
from jax import lax
from jax.experimental import pallas as pl
from jax.experimental.pallas import tpu as pltpu
from jax.experimental.pallas import tpu_sc as plsc

D_MODEL = 1024
_NUM_CORES = 2
_NUM_SUBCORES = 16
_NUM_WORKERS = _NUM_CORES * _NUM_SUBCORES


@functools.lru_cache(maxsize=None)
def _build_mpmd_kernel(
    B: int,
    scs_rows: int,
    chunk: int,
    nbuf: int,
    lag: int,
    s_chunk: int,
):
    tec_rows = B - scs_rows
    rows_per_worker = tec_rows // _NUM_WORKERS
    n_chunks = rows_per_worker // chunk
    rows_per_scs = scs_rows // _NUM_CORES
    n_s_chunks = rows_per_scs // s_chunk

    vmesh = plsc.VectorSubcoreMesh(core_axis_name="c", subcore_axis_name="s")
    smesh = plsc.ScalarSubcoreMesh(axis_name="c", num_cores=_NUM_CORES)

    def tec_fn(toks_hbm, table_hbm, out_hbm, idx_v, buf_v, gsems, osems,
               idx_s, buf_s, s_gsem, s_osem):
        del idx_s, buf_s, s_gsem, s_osem
        wid = lax.axis_index("s") * _NUM_CORES + lax.axis_index("c")
        base = wid * rows_per_worker
        pltpu.sync_copy(toks_hbm.at[pl.ds(base, rows_per_worker)], idx_v)

        def gather_copy(ci, slot):
            return pltpu.make_async_copy(
                table_hbm.at[idx_v.at[pl.ds(ci * chunk, chunk)]],
                buf_v.at[slot],
                gsems[slot],
            )

        def out_copy(ci, slot):
            return pltpu.make_async_copy(
                buf_v.at[slot],
                out_hbm.at[pl.ds(base + ci * chunk, chunk)],
                osems[slot],
            )

        for i in range(n_chunks + lag):
            if i < n_chunks:
                s = i % nbuf
                if i >= nbuf:
                    out_copy(i - nbuf, s).wait()
                gather_copy(i, s).start()
            j = i - lag
            if j >= 0:
                gather_copy(j, j % nbuf).wait()
                out_copy(j, j % nbuf).start()
        for j in range(max(0, n_chunks - nbuf), n_chunks):
            out_copy(j, j % nbuf).wait()

    def scs_fn(toks_hbm, table_hbm, out_hbm, idx_v, buf_v, gsems, osems,
               idx_s, buf_s, s_gsem, s_osem):
        del idx_v, buf_v, gsems, osems
        cid = lax.axis_index("c")
        base = tec_rows + cid * rows_per_scs

        def out_copy(ci, slot):
            return pltpu.make_async_copy(
                buf_s.at[slot],
                out_hbm.at[pl.ds(base + ci * s_chunk, s_chunk)],
                s_osem,
            )

        for ci in range(n_s_chunks):
            slot = ci % 2
            start = base + ci * s_chunk
            if ci >= 2:
                out_copy(ci - 2, slot).wait()
            pltpu.sync_copy(toks_hbm.at[pl.ds(start, s_chunk)], idx_s)

            def row_body(r, slot=slot):
                pltpu.make_async_copy(
                    table_hbm.at[pl.ds(idx_s[r], 1)],
                    buf_s.at[slot].at[pl.ds(r, 1)],
                    s_gsem,
                ).start()

            pl.loop(0, s_chunk)(row_body)
            pltpu.make_async_copy(
                table_hbm.at[pl.ds(0, s_chunk)], buf_s.at[slot], s_gsem
            ).wait()
            out_copy(ci, slot).start()
        for ci in range(max(0, n_s_chunks - 2), n_s_chunks):
            out_copy(ci, ci % 2).wait()

    return pl.kernel(
        body=[tec_fn, scs_fn],
        mesh=[vmesh, smesh],
        out_type=jax.ShapeDtypeStruct((B, D_MODEL), jnp.float32),
        scratch_types=[
            (pltpu.VMEM @ vmesh)((rows_per_worker,), jnp.int32),
            (pltpu.VMEM @ vmesh)((nbuf, chunk, D_MODEL), jnp.float32),
            tuple((pltpu.SemaphoreType.DMA @ vmesh) for _ in range(nbuf)),
            tuple((pltpu.SemaphoreType.DMA @ vmesh) for _ in range(nbuf)),
            (pltpu.SMEM @ smesh)((s_chunk,), jnp.int32),
            pltpu.VMEM_SHARED((2, s_chunk, D_MODEL), jnp.float32),
            pltpu.SemaphoreType.DMA @ smesh,
            pltpu.SemaphoreType.DMA @ smesh,
        ],
    )


def kernel(toks, W_E):
    n_batch, seq = toks.shape
    B = n_batch * seq
    flat = toks.reshape(B).astype(jnp.int32)
    out = _build_mpmd_kernel(
        B, scs_rows=4096, chunk=16, nbuf=3, lag=2, s_chunk=512
    )(flat, W_E)
    return out.reshape(n_batch, seq, D_MODEL)
